# baseline (speedup 1.0000x reference)
import functools

import jax
import jax.numpy as jnp
from jax import lax
from jax.experimental import pallas as pl
from jax.experimental.pallas import tpu as pltpu
from jax.experimental.pallas import tpu_sc as plsc

_EMBED_DIM = 5
_N_CAT = 47
_BATCH = 16384

_NC = 2
_NS = 16
_NW = _NC * _NS
_B_PER_W = _BATCH // _NW
_CHUNK = 128
_NCHUNK = _B_PER_W // _CHUNK
_D_PAD = 48


def _fuse_body(emb_ref, w_ref, b_ref, out_ref):
    fused = (
        jnp.dot(emb_ref[...], w_ref[...], preferred_element_type=jnp.float32)
        + b_ref[...]
    )
    out_ref[...] = jnp.pad(fused, ((0, 0), (0, _D_PAD - _N_CAT)))


def _fuse_table(emb_table, dense_kernel, dense_bias):
    return pl.pallas_call(
        _fuse_body,
        out_shape=jax.ShapeDtypeStruct((_N_CAT, _D_PAD), jnp.float32),
    )(emb_table, dense_kernel, dense_bias.reshape(1, _N_CAT))


_sc_mesh = plsc.VectorSubcoreMesh(core_axis_name="c", subcore_axis_name="s")


@functools.partial(
    pl.kernel,
    out_type=jax.ShapeDtypeStruct((_BATCH, _D_PAD), jnp.float32),
    mesh=_sc_mesh,
    scratch_types=[
        pltpu.VMEM((_NCHUNK, _CHUNK), jnp.int32),
        pltpu.VMEM((_B_PER_W, _D_PAD), jnp.float32),
        pltpu.SemaphoreType.DMA,
    ],
    compiler_params=pltpu.CompilerParams(use_tc_tiling_on_sc=False),
)
def _sc_gather(table_hbm, idx_hbm, out_hbm, idx_v, rows_v, sem):
    wid = lax.axis_index("s") * _NC + lax.axis_index("c")
    pltpu.sync_copy(idx_hbm.at[wid], idx_v)
    copies = [
        pltpu.async_copy(
            table_hbm.at[idx_v.at[j]],
            rows_v.at[pl.ds(j * _CHUNK, _CHUNK)],
            sem,
        )
        for j in range(_NCHUNK)
    ]
    for cp in copies:
        cp.wait()
    pltpu.sync_copy(rows_v, out_hbm.at[pl.ds(wid * _B_PER_W, _B_PER_W)])


def kernel(inputs, emb_table, dense_kernel, dense_bias):
    fused = _fuse_table(emb_table, dense_kernel, dense_bias)
    idx = inputs.reshape(_NW, _NCHUNK, _CHUNK)
    out = _sc_gather(fused, idx)
    return out[:, : _N_CAT].reshape(_BATCH, 1, _N_CAT)


# speedup vs baseline: 1.0321x; 1.0321x over previous
import functools

import jax
import jax.numpy as jnp
from jax import lax
from jax.experimental import pallas as pl
from jax.experimental.pallas import tpu as pltpu
from jax.experimental.pallas import tpu_sc as plsc

_EMBED_DIM = 5
_N_CAT = 47
_BATCH = 16384

_NC = 2
_NS = 16
_NW = _NC * _NS
_B_PER_W = _BATCH // _NW
_D_PAD = 48
_L = 16
_NBLK = _B_PER_W // _L

_sc_mesh = plsc.VectorSubcoreMesh(core_axis_name="c", subcore_axis_name="s")


@functools.partial(
    pl.kernel,
    out_type=jax.ShapeDtypeStruct((_BATCH * _N_CAT,), jnp.float32),
    mesh=_sc_mesh,
    scratch_types=[
        pltpu.VMEM((_B_PER_W,), jnp.int32),
        pltpu.VMEM((_N_CAT * _EMBED_DIM + _L,), jnp.float32),
        pltpu.VMEM((16 * _L,), jnp.float32),
        pltpu.VMEM((_D_PAD,), jnp.float32),
        pltpu.VMEM((_N_CAT * _D_PAD,), jnp.float32),
        pltpu.VMEM((_B_PER_W * _N_CAT,), jnp.float32),
        pltpu.SemaphoreType.DMA,
    ],
    compiler_params=pltpu.CompilerParams(
        use_tc_tiling_on_sc=False, needs_layout_passes=False
    ),
)
def _sc_fused_lookup(
    idx_hbm, emb_hbm, w_hbm, b_hbm, out_hbm,
    idx_v, emb_v, w_v, bias_v, table_v, out_v, sem,
):
    wid = lax.axis_index("s") * _NC + lax.axis_index("c")

    idx_cp = pltpu.async_copy(
        idx_hbm.at[pl.ds(wid * _B_PER_W, _B_PER_W)], idx_v, sem
    )
    pltpu.sync_copy(emb_hbm, emb_v.at[pl.ds(0, _N_CAT * _EMBED_DIM)])
    pltpu.sync_copy(w_hbm, w_v.at[pl.ds(0, _EMBED_DIM * _N_CAT)])
    pltpu.sync_copy(b_hbm, bias_v.at[pl.ds(0, _N_CAT)])

    w_vregs = [
        [w_v[pl.ds(e * _N_CAT + k * _L, _L)] for k in range(3)]
        for e in range(_EMBED_DIM)
    ]
    b_vregs = [bias_v[pl.ds(k * _L, _L)] for k in range(3)]

    def table_row(r, _):
        accs = list(b_vregs)
        erow = emb_v[pl.ds(r * _EMBED_DIM, _L)]
        for e in range(_EMBED_DIM):
            s = erow[e]
            for k in range(3):
                accs[k] = accs[k] + s * w_vregs[e][k]
        for k in range(3):
            table_v[pl.ds(r * _D_PAD + k * _L, _L)] = accs[k]
        return _

    lax.fori_loop(0, _N_CAT, table_row, 0)
    idx_cp.wait()

    iota = lax.iota(jnp.int32, _L)

    def gather_block(i, _):
        b0 = i * _L
        idx16 = idx_v[pl.ds(b0, _L)]
        src = idx16 * _D_PAD
        dst = (b0 + iota) * _N_CAT
        for j in range(_N_CAT):
            vals = plsc.load_gather(table_v, [src])
            plsc.store_scatter(out_v, [dst], vals)
            if j != _N_CAT - 1:
                src = src + 1
                dst = dst + 1
        return _

    lax.fori_loop(0, _NBLK, gather_block, 0)

    pltpu.sync_copy(
        out_v,
        out_hbm.at[pl.ds(wid * (_B_PER_W * _N_CAT), _B_PER_W * _N_CAT)],
    )


def kernel(inputs, emb_table, dense_kernel, dense_bias):
    out = _sc_fused_lookup(
        inputs.reshape(_BATCH),
        emb_table.reshape(_N_CAT * _EMBED_DIM),
        dense_kernel.reshape(_EMBED_DIM * _N_CAT),
        dense_bias,
    )
    return out.reshape(_BATCH, 1, _N_CAT)
